# Initial kernel scaffold; baseline (speedup 1.0000x reference)
import jax
import jax.numpy as jnp
from jax.experimental import pallas as pl


def kernel(init_sim, prev_sim, init_seg, prev_seg):
    raise NotImplementedError("write your pallas kernel here")


# speedup vs baseline: 7.2919x; 7.2919x over previous
import functools

import jax
import jax.numpy as jnp
from jax import lax
from jax.experimental import pallas as pl
from jax.experimental.pallas import tpu as pltpu
from jax.experimental.pallas import tpu_sc as plsc

L = 16
NC = 2
NS = 16
NW = NC * NS


def _sc_matcher(B, HW, rows_per_w, blk):
    nvr = HW // L
    nblk = rows_per_w // blk
    mesh = plsc.VectorSubcoreMesh(core_axis_name="c", subcore_axis_name="s")

    @functools.partial(
        pl.kernel,
        out_type=jax.ShapeDtypeStruct((4 * B * NS * HW,), jnp.float32),
        mesh=mesh,
        compiler_params=pltpu.CompilerParams(needs_layout_passes=False),
        scratch_types=[
            pltpu.VMEM((blk * HW,), jnp.float32),
            pltpu.VMEM((4 * HW,), jnp.float32),
            pltpu.VMEM((4 * rows_per_w * L,), jnp.float32),
        ],
    )
    def body(prev_hbm, init_hbm, wpb_hbm, wpf_hbm, wib_hbm, wif_hbm,
             out_hbm, rowbuf, acc, wbuf):
        wid = lax.axis_index("s") * NC + lax.axis_index("c")
        b = wid // NS
        k = wid % NS
        r0 = wid * rows_per_w

        zeros = jnp.zeros((L,), jnp.float32)
        ones = jnp.full((L,), 1.0, jnp.float32)
        neg = jnp.full((L,), -jnp.inf, jnp.float32)
        pos = jnp.full((L,), jnp.inf, jnp.float32)

        def zero_acc(j, _):
            acc[pl.ds(j * L, L)] = zeros
            return 0
        lax.fori_loop(0, 4 * nvr, zero_acc, 0)

        for seg, w_hbm in enumerate((wpb_hbm, wpf_hbm, wib_hbm, wif_hbm)):
            pltpu.sync_copy(w_hbm.at[pl.ds(r0 * L, rows_per_w * L)],
                            wbuf.at[pl.ds(seg * rows_per_w * L, rows_per_w * L)])

        def prev_blk(bi, carry):
            mvb, mvf = carry
            pltpu.sync_copy(
                prev_hbm.at[pl.ds((r0 + bi * blk) * HW, blk * HW)], rowbuf)

            def row_body(rr, carry2):
                mvb, mvf = carry2
                rbase = rr * HW

                def p1(j, c):
                    t1, t2, t3, t4, rmin = c
                    v = rowbuf[pl.ds(rbase + j * L, L)]
                    lo = jnp.minimum(t1, v); t1 = jnp.maximum(t1, v)
                    lo2 = jnp.minimum(t2, lo); t2 = jnp.maximum(t2, lo)
                    lo3 = jnp.minimum(t3, lo2); t3 = jnp.maximum(t3, lo2)
                    t4 = jnp.maximum(t4, lo3)
                    rmin = jnp.minimum(rmin, v)
                    return (t1, t2, t3, t4, rmin)

                t1, t2, t3, t4, rmin = lax.fori_loop(
                    0, nvr, p1, (neg, neg, neg, neg, pos))
                ts = (t1, t2, t3, t4)

                def count_eq(m_s):
                    mb = jnp.full((L,), m_s)
                    tot = jnp.float32(0.0)
                    for t in ts:
                        tot = tot + jnp.sum(jnp.where(t == mb, ones, zeros))
                    return tot

                def next_max(m_s):
                    mb = jnp.full((L,), m_s)
                    cur = neg
                    for t in ts:
                        cur = jnp.maximum(cur, jnp.where(t < mb, t, neg))
                    return jnp.max(cur)

                m1 = jnp.max(t1)
                c1 = count_eq(m1)
                m2 = next_max(m1)
                c2 = count_eq(m2)
                m3 = next_max(m2)
                c3 = count_eq(m3)
                m4 = next_max(m3)
                four = jnp.float32(4.0)
                xcut = jnp.where(
                    c1 >= four, m1,
                    jnp.where(c1 + c2 >= four, m2,
                              jnp.where(c1 + c2 + c3 >= four, m3, m4)))
                mn_s = jnp.min(rmin)

                widx = (bi * blk + rr) * L
                wb = wbuf[pl.ds(widx, L)]
                wf = wbuf[pl.ds(HW + widx, L)]
                mnv = jnp.full((L,), mn_s)
                mvb = jnp.maximum(mvb, wb * mnv)
                mvf = jnp.maximum(mvf, wf * mnv)

                xc = jnp.full((L,), xcut)

                def p2(j, _):
                    v = rowbuf[pl.ds(rbase + j * L, L)]
                    xm = jnp.where(v >= xc, v, zeros)
                    ab = acc[pl.ds(2 * HW + j * L, L)]
                    acc[pl.ds(2 * HW + j * L, L)] = jnp.maximum(ab, xm * wb)
                    af = acc[pl.ds(3 * HW + j * L, L)]
                    acc[pl.ds(3 * HW + j * L, L)] = jnp.maximum(af, xm * wf)
                    return 0
                lax.fori_loop(0, nvr, p2, 0)
                return (mvb, mvf)

            return lax.fori_loop(0, blk, row_body, (mvb, mvf))

        mvb, mvf = lax.fori_loop(0, nblk, prev_blk, (neg, neg))

        def init_blk(bi, _):
            pltpu.sync_copy(
                init_hbm.at[pl.ds((r0 + bi * blk) * HW, blk * HW)], rowbuf)

            def row_body(rr, _):
                rbase = rr * HW
                widx = (bi * blk + rr) * L
                wb = wbuf[pl.ds(2 * HW + widx, L)]
                wf = wbuf[pl.ds(3 * HW + widx, L)]

                def pg(j, _):
                    v = rowbuf[pl.ds(rbase + j * L, L)]
                    ab = acc[pl.ds(j * L, L)]
                    acc[pl.ds(j * L, L)] = jnp.maximum(ab, v * wb)
                    af = acc[pl.ds(HW + j * L, L)]
                    acc[pl.ds(HW + j * L, L)] = jnp.maximum(af, v * wf)
                    return 0
                lax.fori_loop(0, nvr, pg, 0)
                return 0

            lax.fori_loop(0, blk, row_body, 0)
            return 0
        lax.fori_loop(0, nblk, init_blk, 0)

        mbv = jnp.full((L,), jnp.max(mvb))
        mfv = jnp.full((L,), jnp.max(mvf))

        def clamp(j, _):
            acc[pl.ds(2 * HW + j * L, L)] = jnp.maximum(
                acc[pl.ds(2 * HW + j * L, L)], mbv)
            acc[pl.ds(3 * HW + j * L, L)] = jnp.maximum(
                acc[pl.ds(3 * HW + j * L, L)], mfv)
            return 0
        lax.fori_loop(0, nvr, clamp, 0)

        for ci in range(4):
            q = b * 4 + ci
            pltpu.sync_copy(acc.at[pl.ds(ci * HW, HW)],
                            out_hbm.at[pl.ds((q * NS + k) * HW, HW)])

    return body


def _tc_reduce(q, nw, hw):
    def body(x_ref, o_ref):
        o_ref[...] = jnp.max(x_ref[...], axis=1)

    return pl.pallas_call(
        body,
        out_shape=jax.ShapeDtypeStruct((q, hw), jnp.float32),
    )


def kernel(init_sim, prev_sim, init_seg, prev_seg):
    B, HW, H, W = init_sim.shape
    rows = B * HW
    rows_per_w = rows // NW
    blk = 16

    prev_flat = prev_sim.reshape(rows * HW)
    init_flat = init_sim.reshape(rows * HW)

    def expand(seg_ch):
        return jnp.broadcast_to(
            seg_ch.reshape(rows, 1), (rows, L)).reshape(rows * L)

    wpb = expand(prev_seg[:, 0])
    wpf = expand(prev_seg[:, 1])
    wib = expand(init_seg[:, 0])
    wif = expand(init_seg[:, 1])

    sc = _sc_matcher(B, HW, rows_per_w, blk)
    part = sc(prev_flat, init_flat, wpb, wpf, wib, wif)
    part = part.reshape(4 * B, NS, HW)
    out = _tc_reduce(4 * B, NS, HW)(part)
    return out.reshape(B, 4, H, W)


# speedup vs baseline: 13.5557x; 1.8590x over previous
import functools

import jax
import jax.numpy as jnp
from jax import lax
from jax.experimental import pallas as pl
from jax.experimental.pallas import tpu as pltpu
from jax.experimental.pallas import tpu_sc as plsc

L = 16
NC = 2
NS = 16
NW = NC * NS
U = 4
G = 4


def _merge4(a, b):
    z1 = jnp.maximum(a[0], b[3])
    z2 = jnp.maximum(a[1], b[2])
    z3 = jnp.maximum(a[2], b[1])
    z4 = jnp.maximum(a[3], b[0])
    w1 = jnp.maximum(z1, z3); w3 = jnp.minimum(z1, z3)
    w2 = jnp.maximum(z2, z4); w4 = jnp.minimum(z2, z4)
    s1 = jnp.maximum(w1, w2); s2 = jnp.minimum(w1, w2)
    s3 = jnp.maximum(w3, w4); s4 = jnp.minimum(w3, w4)
    return (s1, s2, s3, s4)


def _sc_matcher(B, HW, rows_per_w, blk):
    nvr = HW // L
    nblk = rows_per_w // blk
    blkw = blk * HW
    mesh = plsc.VectorSubcoreMesh(core_axis_name="c", subcore_axis_name="s")

    @functools.partial(
        pl.kernel,
        out_type=jax.ShapeDtypeStruct((4 * B * NS * HW,), jnp.float32),
        mesh=mesh,
        compiler_params=pltpu.CompilerParams(needs_layout_passes=False),
        scratch_types=[
            pltpu.VMEM((blkw,), jnp.float32),
            pltpu.VMEM((blkw,), jnp.float32),
            pltpu.VMEM((4 * HW,), jnp.float32),
            pltpu.VMEM((4 * rows_per_w * L,), jnp.float32),
            pltpu.VMEM((blk * L,), jnp.float32),
            pltpu.VMEM((2 * L,), jnp.float32),
            pltpu.SemaphoreType.DMA,
            pltpu.SemaphoreType.DMA,
        ],
    )
    def body(prev_hbm, init_hbm, wpb_hbm, wpf_hbm, wib_hbm, wif_hbm,
             out_hbm, rowbuf0, rowbuf1, acc, wbuf, xcbuf, mbuf, sem0, sem1):
        rowbufs = (rowbuf0, rowbuf1)
        wid = lax.axis_index("s") * NC + lax.axis_index("c")
        b = wid // NS
        k = wid % NS
        r0 = wid * rows_per_w
        sems = (sem0, sem1)

        zeros = jnp.zeros((L,), jnp.float32)
        ones = jnp.full((L,), 1.0, jnp.float32)
        neg = jnp.full((L,), -jnp.inf, jnp.float32)
        pos = jnp.full((L,), jnp.inf, jnp.float32)

        def zero_acc(j, _):
            acc[pl.ds(j * L, L)] = zeros
            return 0
        lax.fori_loop(0, 4 * nvr, zero_acc, 0)
        mbuf[pl.ds(0, L)] = zeros
        mbuf[pl.ds(L, L)] = zeros

        for seg, w_hbm in enumerate((wpb_hbm, wpf_hbm, wib_hbm, wif_hbm)):
            pltpu.sync_copy(w_hbm.at[pl.ds(r0 * L, rows_per_w * L)],
                            wbuf.at[pl.ds(seg * rows_per_w * L, rows_per_w * L)])

        def start_dma(arr_hbm, bi, p):
            pltpu.async_copy(
                arr_hbm.at[pl.ds((r0 + bi * blk) * HW, blkw)],
                rowbufs[p], sems[p])

        def wait_dma(p):
            pltpu.make_async_copy(
                prev_hbm.at[pl.ds(0, blkw)], rowbufs[p], sems[p]).wait()

        start_dma(prev_hbm, 0, 0)
        start_dma(prev_hbm, 1, 1)

        def prev_outer(h, _):
            for p in range(2):
                bi = h * 2 + p
                wait_dma(p)
                rb = rowbufs[p]

                def rowA(rr, _):
                    rbase = rr * HW

                    def p1(jj, c):
                        ts0, ts1, ts2, ts3, rmin = c
                        tss = [ts0, ts1, ts2, ts3]
                        vs = []
                        for u in range(U):
                            v = rb[pl.ds(rbase + (jj * U + u) * L, L)]
                            vs.append(v)
                            t1, t2, t3, t4 = tss[u]
                            lo = jnp.minimum(t1, v); t1 = jnp.maximum(t1, v)
                            lo2 = jnp.minimum(t2, lo); t2 = jnp.maximum(t2, lo)
                            lo3 = jnp.minimum(t3, lo2); t3 = jnp.maximum(t3, lo2)
                            t4 = jnp.maximum(t4, lo3)
                            tss[u] = (t1, t2, t3, t4)
                        m01 = jnp.minimum(vs[0], vs[1])
                        m23 = jnp.minimum(vs[2], vs[3])
                        rmin = jnp.minimum(rmin, jnp.minimum(m01, m23))
                        return (tss[0], tss[1], tss[2], tss[3], rmin)

                    t0 = (neg, neg, neg, neg)
                    ts0, ts1, ts2, ts3, rmin = lax.fori_loop(
                        0, nvr // U, p1, (t0, t0, t0, t0, pos))
                    ts = _merge4(_merge4(ts0, ts1), _merge4(ts2, ts3))

                    def count_eq(m_s):
                        mb = jnp.full((L,), m_s)
                        tot = jnp.float32(0.0)
                        for t in ts:
                            tot = tot + jnp.sum(jnp.where(t == mb, ones, zeros))
                        return tot

                    def next_max(m_s):
                        mb = jnp.full((L,), m_s)
                        cur = neg
                        for t in ts:
                            cur = jnp.maximum(cur, jnp.where(t < mb, t, neg))
                        return jnp.max(cur)

                    m1 = jnp.max(ts[0])
                    c1 = count_eq(m1)
                    m2 = next_max(m1)
                    c2 = count_eq(m2)
                    m3 = next_max(m2)
                    c3 = count_eq(m3)
                    m4 = next_max(m3)
                    four = jnp.float32(4.0)
                    xcut = jnp.where(
                        c1 >= four, m1,
                        jnp.where(c1 + c2 >= four, m2,
                                  jnp.where(c1 + c2 + c3 >= four, m3, m4)))
                    xcbuf[pl.ds(rr * L, L)] = jnp.full((L,), xcut)

                    mnv = jnp.full((L,), jnp.min(rmin))
                    widx = (bi * blk + rr) * L
                    wb = wbuf[pl.ds(widx, L)]
                    wf = wbuf[pl.ds(rows_per_w * L + widx, L)]
                    mbuf[pl.ds(0, L)] = jnp.maximum(mbuf[pl.ds(0, L)], wb * mnv)
                    mbuf[pl.ds(L, L)] = jnp.maximum(mbuf[pl.ds(L, L)], wf * mnv)
                    return 0
                lax.fori_loop(0, blk, rowA, 0)

                for g in range(blk // G):
                    xcs, wbs, wfs = [], [], []
                    for r in range(G):
                        row = g * G + r
                        widx = (bi * blk + row) * L
                        xcs.append(xcbuf[pl.ds(row * L, L)])
                        wbs.append(wbuf[pl.ds(widx, L)])
                        wfs.append(wbuf[pl.ds(rows_per_w * L + widx, L)])

                    def pB(j, _):
                        ab = acc[pl.ds(2 * HW + j * L, L)]
                        af = acc[pl.ds(3 * HW + j * L, L)]
                        for r in range(G):
                            v = rb[pl.ds((g * G + r) * HW + j * L, L)]
                            xm = jnp.where(v >= xcs[r], v, zeros)
                            ab = jnp.maximum(ab, xm * wbs[r])
                            af = jnp.maximum(af, xm * wfs[r])
                        acc[pl.ds(2 * HW + j * L, L)] = ab
                        acc[pl.ds(3 * HW + j * L, L)] = af
                        return 0
                    lax.fori_loop(0, nvr, pB, 0)

                @pl.when(bi + 2 < nblk)
                def _():
                    start_dma(prev_hbm, bi + 2, p)
            return 0
        lax.fori_loop(0, nblk // 2, prev_outer, 0)

        start_dma(init_hbm, 0, 0)
        start_dma(init_hbm, 1, 1)

        def init_outer(h, _):
            for p in range(2):
                bi = h * 2 + p
                wait_dma(p)
                rb = rowbufs[p]
                for g in range(blk // G):
                    wbs, wfs = [], []
                    for r in range(G):
                        widx = (bi * blk + g * G + r) * L
                        wbs.append(wbuf[pl.ds(2 * rows_per_w * L + widx, L)])
                        wfs.append(wbuf[pl.ds(3 * rows_per_w * L + widx, L)])

                    def pG(j, _):
                        ab = acc[pl.ds(j * L, L)]
                        af = acc[pl.ds(HW + j * L, L)]
                        for r in range(G):
                            v = rb[pl.ds((g * G + r) * HW + j * L, L)]
                            ab = jnp.maximum(ab, v * wbs[r])
                            af = jnp.maximum(af, v * wfs[r])
                        acc[pl.ds(j * L, L)] = ab
                        acc[pl.ds(HW + j * L, L)] = af
                        return 0
                    lax.fori_loop(0, nvr, pG, 0)

                @pl.when(bi + 2 < nblk)
                def _():
                    start_dma(init_hbm, bi + 2, p)
            return 0
        lax.fori_loop(0, nblk // 2, init_outer, 0)

        mbv = jnp.full((L,), jnp.max(mbuf[pl.ds(0, L)]))
        mfv = jnp.full((L,), jnp.max(mbuf[pl.ds(L, L)]))

        def clamp(j, _):
            acc[pl.ds(2 * HW + j * L, L)] = jnp.maximum(
                acc[pl.ds(2 * HW + j * L, L)], mbv)
            acc[pl.ds(3 * HW + j * L, L)] = jnp.maximum(
                acc[pl.ds(3 * HW + j * L, L)], mfv)
            return 0
        lax.fori_loop(0, nvr, clamp, 0)

        for ci in range(4):
            q = b * 4 + ci
            pltpu.sync_copy(acc.at[pl.ds(ci * HW, HW)],
                            out_hbm.at[pl.ds((q * NS + k) * HW, HW)])

    return body


def _tc_reduce(q, hw):
    def body(x_ref, o_ref):
        o_ref[...] = jnp.max(x_ref[...], axis=1)

    return pl.pallas_call(
        body,
        out_shape=jax.ShapeDtypeStruct((q, hw), jnp.float32),
    )


def kernel(init_sim, prev_sim, init_seg, prev_seg):
    B, HW, H, W = init_sim.shape
    rows = B * HW
    rows_per_w = rows // NW
    blk = 12

    prev_flat = prev_sim.reshape(rows * HW)
    init_flat = init_sim.reshape(rows * HW)

    def expand(seg_ch):
        return jnp.broadcast_to(
            seg_ch.reshape(rows, 1), (rows, L)).reshape(rows * L)

    wpb = expand(prev_seg[:, 0])
    wpf = expand(prev_seg[:, 1])
    wib = expand(init_seg[:, 0])
    wif = expand(init_seg[:, 1])

    sc = _sc_matcher(B, HW, rows_per_w, blk)
    part = sc(prev_flat, init_flat, wpb, wpf, wib, wif)
    part = part.reshape(4 * B, NS, HW)
    out = _tc_reduce(4 * B, HW)(part)
    return out.reshape(B, 4, H, W)


# speedup vs baseline: 21.1294x; 1.5587x over previous
import functools

import jax
import jax.numpy as jnp
from jax import lax
from jax.experimental import pallas as pl
from jax.experimental.pallas import tpu as pltpu
from jax.experimental.pallas import tpu_sc as plsc

L = 16
NC = 2
NS = 16
NW = NC * NS
U = 4
G = 4


def _merge4(a, b):
    z1 = jnp.maximum(a[0], b[3])
    z2 = jnp.maximum(a[1], b[2])
    z3 = jnp.maximum(a[2], b[1])
    z4 = jnp.maximum(a[3], b[0])
    w1 = jnp.maximum(z1, z3); w3 = jnp.minimum(z1, z3)
    w2 = jnp.maximum(z2, z4); w4 = jnp.minimum(z2, z4)
    s1 = jnp.maximum(w1, w2); s2 = jnp.minimum(w1, w2)
    s3 = jnp.maximum(w3, w4); s4 = jnp.minimum(w3, w4)
    return (s1, s2, s3, s4)


def _sc_matcher(B, HW, rows_per_w, blk):
    nvr = HW // L
    nblk = rows_per_w // blk
    mesh = plsc.VectorSubcoreMesh(core_axis_name="c", subcore_axis_name="s")

    @functools.partial(
        pl.kernel,
        out_type=jax.ShapeDtypeStruct((4 * B * NS * HW,), jnp.float32),
        mesh=mesh,
        compiler_params=pltpu.CompilerParams(needs_layout_passes=False),
        scratch_types=[
            pltpu.VMEM((blk, HW), jnp.float32),
            pltpu.VMEM((blk, HW), jnp.float32),
            pltpu.VMEM((4 * HW,), jnp.float32),
            pltpu.VMEM((4 * rows_per_w * L,), jnp.float32),
            pltpu.VMEM((blk * L,), jnp.float32),
            pltpu.VMEM((2 * L,), jnp.float32),
            pltpu.SemaphoreType.DMA,
            pltpu.SemaphoreType.DMA,
        ],
    )
    def body(prev_hbm, init_hbm, wpb_hbm, wpf_hbm, wib_hbm, wif_hbm,
             out_hbm, rowbuf0, rowbuf1, acc, wbuf, xcbuf, mbuf, sem0, sem1):
        rowbufs = (rowbuf0, rowbuf1)
        wid = lax.axis_index("s") * NC + lax.axis_index("c")
        b = wid // NS
        k = wid % NS
        r0 = wid * rows_per_w
        rb0 = k * rows_per_w
        sems = (sem0, sem1)

        zeros = jnp.zeros((L,), jnp.float32)
        ones = jnp.full((L,), 1.0, jnp.float32)
        neg = jnp.full((L,), -jnp.inf, jnp.float32)
        pos = jnp.full((L,), jnp.inf, jnp.float32)

        def zero_acc(j, _):
            acc[pl.ds(j * L, L)] = zeros
            return 0
        lax.fori_loop(0, 4 * nvr, zero_acc, 0)
        mbuf[pl.ds(0, L)] = zeros
        mbuf[pl.ds(L, L)] = zeros

        for seg, w_hbm in enumerate((wpb_hbm, wpf_hbm, wib_hbm, wif_hbm)):
            pltpu.sync_copy(w_hbm.at[pl.ds(r0 * L, rows_per_w * L)],
                            wbuf.at[pl.ds(seg * rows_per_w * L, rows_per_w * L)])

        def start_dma(arr_hbm, bi, p):
            pltpu.async_copy(
                arr_hbm.at[b, pl.ds(rb0 + bi * blk, blk)],
                rowbufs[p], sems[p])

        def wait_dma(p):
            pltpu.make_async_copy(
                prev_hbm.at[0, pl.ds(0, blk)], rowbufs[p], sems[p]).wait()

        start_dma(prev_hbm, 0, 0)
        start_dma(prev_hbm, 1, 1)

        def prev_outer(h, _):
            for p in range(2):
                bi = h * 2 + p
                wait_dma(p)
                rb = rowbufs[p]

                def rowA(rr, _):
                    def p1(jj, c):
                        ts0, ts1, ts2, ts3, rmin = c
                        tss = [ts0, ts1, ts2, ts3]
                        vs = []
                        for u in range(U):
                            v = rb[rr, pl.ds((jj * U + u) * L, L)]
                            vs.append(v)
                            t1, t2, t3, t4 = tss[u]
                            lo = jnp.minimum(t1, v); t1 = jnp.maximum(t1, v)
                            lo2 = jnp.minimum(t2, lo); t2 = jnp.maximum(t2, lo)
                            lo3 = jnp.minimum(t3, lo2); t3 = jnp.maximum(t3, lo2)
                            t4 = jnp.maximum(t4, lo3)
                            tss[u] = (t1, t2, t3, t4)
                        m01 = jnp.minimum(vs[0], vs[1])
                        m23 = jnp.minimum(vs[2], vs[3])
                        rmin = jnp.minimum(rmin, jnp.minimum(m01, m23))
                        return (tss[0], tss[1], tss[2], tss[3], rmin)

                    t0 = (neg, neg, neg, neg)
                    ts0, ts1, ts2, ts3, rmin = lax.fori_loop(
                        0, nvr // U, p1, (t0, t0, t0, t0, pos))
                    ts = _merge4(_merge4(ts0, ts1), _merge4(ts2, ts3))

                    def count_eq(m_s):
                        mb = jnp.full((L,), m_s)
                        tot = jnp.float32(0.0)
                        for t in ts:
                            tot = tot + jnp.sum(jnp.where(t == mb, ones, zeros))
                        return tot

                    def next_max(m_s):
                        mb = jnp.full((L,), m_s)
                        cur = neg
                        for t in ts:
                            cur = jnp.maximum(cur, jnp.where(t < mb, t, neg))
                        return jnp.max(cur)

                    m1 = jnp.max(ts[0])
                    c1 = count_eq(m1)
                    m2 = next_max(m1)
                    c2 = count_eq(m2)
                    m3 = next_max(m2)
                    c3 = count_eq(m3)
                    m4 = next_max(m3)
                    four = jnp.float32(4.0)
                    xcut = jnp.where(
                        c1 >= four, m1,
                        jnp.where(c1 + c2 >= four, m2,
                                  jnp.where(c1 + c2 + c3 >= four, m3, m4)))
                    xcbuf[pl.ds(rr * L, L)] = jnp.full((L,), xcut)

                    mnv = jnp.full((L,), jnp.min(rmin))
                    widx = (bi * blk + rr) * L
                    wb = wbuf[pl.ds(widx, L)]
                    wf = wbuf[pl.ds(rows_per_w * L + widx, L)]
                    mbuf[pl.ds(0, L)] = jnp.maximum(mbuf[pl.ds(0, L)], wb * mnv)
                    mbuf[pl.ds(L, L)] = jnp.maximum(mbuf[pl.ds(L, L)], wf * mnv)
                    return 0
                lax.fori_loop(0, blk, rowA, 0)

                for g in range(blk // G):
                    xcs, wbs, wfs = [], [], []
                    for r in range(G):
                        row = g * G + r
                        widx = (bi * blk + row) * L
                        xcs.append(xcbuf[pl.ds(row * L, L)])
                        wbs.append(wbuf[pl.ds(widx, L)])
                        wfs.append(wbuf[pl.ds(rows_per_w * L + widx, L)])

                    def pB(j, _):
                        ab = acc[pl.ds(2 * HW + j * L, L)]
                        af = acc[pl.ds(3 * HW + j * L, L)]
                        for r in range(G):
                            v = rb[g * G + r, pl.ds(j * L, L)]
                            xm = jnp.where(v >= xcs[r], v, zeros)
                            ab = jnp.maximum(ab, xm * wbs[r])
                            af = jnp.maximum(af, xm * wfs[r])
                        acc[pl.ds(2 * HW + j * L, L)] = ab
                        acc[pl.ds(3 * HW + j * L, L)] = af
                        return 0
                    lax.fori_loop(0, nvr, pB, 0)

                @pl.when(bi + 2 < nblk)
                def _():
                    start_dma(prev_hbm, bi + 2, p)
            return 0
        lax.fori_loop(0, nblk // 2, prev_outer, 0)

        start_dma(init_hbm, 0, 0)
        start_dma(init_hbm, 1, 1)

        def init_outer(h, _):
            for p in range(2):
                bi = h * 2 + p
                wait_dma(p)
                rb = rowbufs[p]
                for g in range(blk // G):
                    wbs, wfs = [], []
                    for r in range(G):
                        widx = (bi * blk + g * G + r) * L
                        wbs.append(wbuf[pl.ds(2 * rows_per_w * L + widx, L)])
                        wfs.append(wbuf[pl.ds(3 * rows_per_w * L + widx, L)])

                    def pG(j, _):
                        ab = acc[pl.ds(j * L, L)]
                        af = acc[pl.ds(HW + j * L, L)]
                        for r in range(G):
                            v = rb[g * G + r, pl.ds(j * L, L)]
                            ab = jnp.maximum(ab, v * wbs[r])
                            af = jnp.maximum(af, v * wfs[r])
                        acc[pl.ds(j * L, L)] = ab
                        acc[pl.ds(HW + j * L, L)] = af
                        return 0
                    lax.fori_loop(0, nvr, pG, 0)

                @pl.when(bi + 2 < nblk)
                def _():
                    start_dma(init_hbm, bi + 2, p)
            return 0
        lax.fori_loop(0, nblk // 2, init_outer, 0)

        mbv = jnp.full((L,), jnp.max(mbuf[pl.ds(0, L)]))
        mfv = jnp.full((L,), jnp.max(mbuf[pl.ds(L, L)]))

        def clamp(j, _):
            acc[pl.ds(2 * HW + j * L, L)] = jnp.maximum(
                acc[pl.ds(2 * HW + j * L, L)], mbv)
            acc[pl.ds(3 * HW + j * L, L)] = jnp.maximum(
                acc[pl.ds(3 * HW + j * L, L)], mfv)
            return 0
        lax.fori_loop(0, nvr, clamp, 0)

        for ci in range(4):
            q = b * 4 + ci
            pltpu.sync_copy(acc.at[pl.ds(ci * HW, HW)],
                            out_hbm.at[pl.ds((q * NS + k) * HW, HW)])

    return body


def _sc_reduce(B, HW, H, W):
    nvr = HW // L
    wpr = W // L
    mesh = plsc.VectorSubcoreMesh(core_axis_name="c", subcore_axis_name="s")

    @functools.partial(
        pl.kernel,
        out_type=jax.ShapeDtypeStruct((B, 4, H, W), jnp.float32),
        mesh=mesh,
        compiler_params=pltpu.CompilerParams(needs_layout_passes=False),
        scratch_types=[
            pltpu.VMEM((NS * HW,), jnp.float32),
            pltpu.VMEM((H, W), jnp.float32),
            pltpu.SemaphoreType.DMA,
        ],
    )
    def body(part_hbm, out_hbm, pbuf, obuf, sem):
        wid = lax.axis_index("s") * NC + lax.axis_index("c")

        @pl.when(wid < 4 * B)
        def _():
            q = wid
            bb = q // 4
            ci = q % 4
            pltpu.async_copy(
                part_hbm.at[pl.ds(q * NS * HW, NS * HW)], pbuf, sem).wait()

            def red(s, _):
                for c in range(wpr):
                    j = s * wpr + c
                    m = pbuf[pl.ds(j * L, L)]
                    for kk in range(1, NS):
                        m = jnp.maximum(m, pbuf[pl.ds(kk * HW + j * L, L)])
                    obuf[s, pl.ds(c * L, L)] = m
                return 0
            lax.fori_loop(0, H, red, 0)
            pltpu.sync_copy(obuf, out_hbm.at[bb, ci])

    return body


def kernel(init_sim, prev_sim, init_seg, prev_seg):
    B, HW, H, W = init_sim.shape
    rows = B * HW
    rows_per_w = rows // NW
    blk = 8

    prev3 = prev_sim.reshape(B, HW, HW)
    init3 = init_sim.reshape(B, HW, HW)

    def expand(seg_ch):
        return jnp.broadcast_to(
            seg_ch.reshape(rows, 1), (rows, L)).reshape(rows * L)

    wpb = expand(prev_seg[:, 0])
    wpf = expand(prev_seg[:, 1])
    wib = expand(init_seg[:, 0])
    wif = expand(init_seg[:, 1])

    part = _sc_matcher(B, HW, rows_per_w, blk)(
        prev3, init3, wpb, wpf, wib, wif)
    return _sc_reduce(B, HW, H, W)(part)
